# baseline (speedup 1.0000x reference)
import functools

import jax
import jax.numpy as jnp
from jax import lax
from jax.experimental import pallas as pl
from jax.experimental.pallas import tpu as pltpu
from jax.experimental.pallas import tpu_sc as plsc

N_ROWS = 16384
N_COLS = 26
EMB = 64
NUM_CORES = 2
NUM_SUBCORES = 16
NW = NUM_CORES * NUM_SUBCORES
ROWS_W = N_ROWS // NW
GROUP = 4
NIDX = GROUP * N_COLS
NCHUNK = ROWS_W // GROUP
NBUF = 8
LAG = 4


@jax.jit
def _gather_sc(idx_grp, table):
    mesh = plsc.VectorSubcoreMesh(
        core_axis_name="c", subcore_axis_name="s",
        num_cores=NUM_CORES, num_subcores=NUM_SUBCORES)

    @functools.partial(
        pl.kernel,
        mesh=mesh,
        out_type=jax.ShapeDtypeStruct((N_ROWS, N_COLS, EMB), jnp.float32),
        scratch_types=[
            pltpu.VMEM((NCHUNK, NIDX), jnp.int32),
            pltpu.VMEM((NBUF, NIDX, EMB), jnp.float32),
            pltpu.SemaphoreType.DMA((NBUF,)),
            pltpu.SemaphoreType.DMA((NBUF,)),
        ],
        compiler_params=pltpu.CompilerParams(use_tc_tiling_on_sc=False),
    )
    def k(idx_hbm, table_hbm, out_hbm, idx_v, rows_v, gsem, wsem):
        wid = lax.axis_index("s") * NUM_CORES + lax.axis_index("c")
        base = wid * ROWS_W
        pltpu.sync_copy(idx_hbm.at[wid], idx_v)

        def wait_writes(b):
            for g in range(GROUP):
                pltpu.make_async_copy(
                    rows_v.at[b, pl.ds(g * N_COLS, N_COLS)],
                    out_hbm.at[base], wsem.at[b]).wait()

        NTOT = NCHUNK + NBUF

        @pl.loop(0, NTOT, step=NBUF)
        def _steps(j0):
            for b in range(NBUF):
                j = j0 + b

                @pl.when(j < NCHUNK)
                def _gather_stage():
                    @pl.when(j >= NBUF)
                    def _reuse_wait():
                        wait_writes(b)
                    pltpu.async_copy(
                        table_hbm.at[idx_v.at[j]],
                        rows_v.at[b], gsem.at[b])

                jw = j - LAG
                bw = (b - LAG) % NBUF

                @pl.when(jnp.logical_and(jw >= 0, jw < NCHUNK))
                def _write_stage():
                    pltpu.make_async_copy(
                        table_hbm.at[idx_v.at[0]],
                        rows_v.at[bw], gsem.at[bw]).wait()
                    for g in range(GROUP):
                        pltpu.async_copy(
                            rows_v.at[bw, pl.ds(g * N_COLS, N_COLS)],
                            out_hbm.at[base + jw * GROUP + g],
                            wsem.at[bw])

        for b in range(NBUF):
            wait_writes(b)

    return k(idx_grp, table)


def kernel(idx, table):
    idx_grp = lax.optimization_barrier(
        idx.astype(jnp.int32).reshape(NW, NCHUNK, NIDX))
    return _gather_sc(idx_grp, table)
